# baseline (speedup 1.0000x reference)
import functools

import jax
import jax.numpy as jnp
from jax import lax
from jax.experimental import pallas as pl
from jax.experimental.pallas import tpu as pltpu
from jax.experimental.pallas import tpu_sc as plsc

H = 128
N = 10000
E = 320000
NF = 2000
FE = 10000
HE = 8000
NUM_LAYERS = 3
BSZ = 128

NC, NS = 2, 16
NW = NC * NS
CHUNK = 128

F32 = jnp.float32


def _round_up(a, m):
  return (a + m - 1) // m * m


def _sc_fused(rounds):
  specs = []
  acc_rows = 0
  for rnd in rounds:
    off = 0
    for (table, gidx, sidx, n_out) in rnd:
      e0 = gidx.shape[0]
      nch = -(-e0 // (NW * CHUNK))
      epad = nch * NW * CHUNK
      n_pad = _round_up(n_out + 8, NS * 8)
      sidx = sidx + off
      if epad != e0:
        pad = epad - e0
        gidx = jnp.concatenate([gidx, jnp.zeros((pad,), jnp.int32)])
        sidx = jnp.concatenate([sidx, jnp.full((pad,), off + n_out,
                                                jnp.int32)])
      specs.append(dict(
          table=table,
          g3=gidx.reshape(NW, nch, CHUNK),
          s3=sidx.reshape(NW, nch, CHUNK),
          nch=nch, n_pad=n_pad, off=off, rz=n_pad // NS))
      off += n_pad
    acc_rows = max(acc_rows, off)
  zeros = jnp.zeros((acc_rows, H), F32)
  mesh = plsc.VectorSubcoreMesh(core_axis_name="c", subcore_axis_name="s",
                                num_cores=NC, num_subcores=NS)
  n_ops_per_round = [len(r) for r in rounds]

  def body(*refs):
    k = 0
    ins = []
    for _ in specs:
      ins.append(refs[k:k + 3])
      k += 3
    z_h = refs[k]
    outs = refs[k + 1:k + 1 + len(specs)]
    gi_v, si_v, rows_v, acc_s, sem = refs[k + 1 + len(specs):]

    cid = lax.axis_index("c")
    sid = lax.axis_index("s")
    wid = cid * NS + sid

    op_i = 0
    for r, n_ops in enumerate(n_ops_per_round):
      rops = list(range(op_i, op_i + n_ops))
      op_i += n_ops
      for oi in rops:
        sp = specs[oi]
        dst = sp["off"] + sid * sp["rz"]
        pltpu.sync_copy(z_h.at[pl.ds(dst, sp["rz"])],
                        acc_s.at[pl.ds(dst, sp["rz"])])
      plsc.subcore_barrier()
      for oi in rops:
        sp = specs[oi]
        table_h, g_h, s_h = ins[oi]

        def step(i, carry, table_h=table_h, g_h=g_h, s_h=s_h):
          pltpu.sync_copy(g_h.at[wid, i], gi_v)
          pltpu.async_copy(table_h.at[gi_v], rows_v, sem).wait()
          pltpu.sync_copy(s_h.at[wid, i], si_v)
          pltpu.sync_copy(rows_v, acc_s.at[si_v], add=True)
          return carry

        lax.fori_loop(0, sp["nch"], step, 0)
      plsc.subcore_barrier()
      for oi in rops:
        sp = specs[oi]
        src = sp["off"] + sid * sp["rz"]
        pltpu.sync_copy(acc_s.at[pl.ds(src, sp["rz"])],
                        outs[oi].at[cid, pl.ds(sid * sp["rz"], sp["rz"])])
      plsc.subcore_barrier()

  call = pl.kernel(
      body,
      out_type=[jax.ShapeDtypeStruct((NC, sp["n_pad"], H), F32)
                for sp in specs],
      mesh=mesh,
      scratch_types=[
          pltpu.VMEM((CHUNK,), jnp.int32),
          pltpu.VMEM((CHUNK,), jnp.int32),
          pltpu.VMEM((CHUNK, H), F32),
          pltpu.VMEM_SHARED((acc_rows, H), F32),
          pltpu.SemaphoreType.DMA,
      ],
  )
  args = []
  for sp in specs:
    args += [sp["table"], sp["g3"], sp["s3"]]
  args.append(zeros)
  res = call(*args)
  return list(res) if isinstance(res, (list, tuple)) else [res]


def _dot(a, b):
  return jnp.dot(a, b, preferred_element_type=F32,
                 precision=lax.Precision.HIGHEST)


def _full(shape):
  return pl.BlockSpec(shape, lambda *i: (0,) * len(shape))


def _tc_add2(p2, n, blk=2000):
  def body(p_r, o_r):
    o_r[...] = p_r[0] + p_r[1]

  return pl.pallas_call(
      body,
      grid=(n // blk,),
      in_specs=[pl.BlockSpec((2, blk, H), lambda i: (0, i, 0))],
      out_specs=pl.BlockSpec((blk, H), lambda i: (i, 0)),
      out_shape=jax.ShapeDtypeStruct((n, H), F32),
  )(p2)


def _tc_prep_atom(hist2, cnt2):
  def body(h_r, c_r, hn_r, ra_r, rf_r):
    hist = h_r[0] + h_r[1]
    deg = jnp.sum(hist, axis=1, keepdims=True)
    ra = 1.0 / jnp.maximum(deg, 1.0)
    hn_r[...] = hist * ra
    ra_r[...] = ra
    rf_r[...] = 1.0 / jnp.maximum(c_r[0] + c_r[1], 1.0)

  return pl.pallas_call(
      body,
      in_specs=[_full((2, N, 16)), _full((2, N, 1))],
      out_specs=[_full((N, 16)), _full((N, 1)), _full((N, 1))],
      out_shape=[
          jax.ShapeDtypeStruct((N, 16), F32),
          jax.ShapeDtypeStruct((N, 1), F32),
          jax.ShapeDtypeStruct((N, 1), F32),
      ],
  )(hist2, cnt2)


def _tc_prep_frag(c2):
  def body(c_r, ra_r, rf_r):
    c = c_r[0] + c_r[1]
    r = 1.0 / jnp.maximum(c, 1.0)
    ra_r[...] = r[:, 0:1]
    rf_r[...] = r[:, 1:2]

  return pl.pallas_call(
      body,
      in_specs=[_full((2, NF, 2))],
      out_specs=[_full((NF, 1)), _full((NF, 1))],
      out_shape=[
          jax.ShapeDtypeStruct((NF, 1), F32),
          jax.ShapeDtypeStruct((NF, 1), F32),
      ],
  )(c2)


def _tc_atom_layer(x, accx, histn, rdega, accf, rdegf, wts, blk=2000):
  def body(x_r, ax_r, hn_r, ra_r, af_r, rf_r, be_r, w1x_r, w1e_r, b1_r,
           w2_r, b2_r, wf1_r, bf1_r, wf2_r, bf2_r, wca_r, wcf_r, bc_r,
           wy_r, by_r, xn_r, y_r):
    xm = (ax_r[0] + ax_r[1]) * ra_r[...]
    em = _dot(hn_r[...], be_r[...])
    h = jax.nn.relu(_dot(xm, w1x_r[...]) + _dot(em, w1e_r[...]) + b1_r[...])
    a2a = jax.nn.relu(_dot(h, w2_r[...]) + b2_r[...])
    fm = (af_r[0] + af_r[1]) * rf_r[...]
    f2a = jax.nn.relu(
        _dot(jax.nn.relu(_dot(fm, wf1_r[...]) + bf1_r[...]), wf2_r[...])
        + bf2_r[...])
    xn = x_r[...] + jax.nn.relu(
        _dot(a2a, wca_r[...]) + _dot(f2a, wcf_r[...]) + bc_r[...])
    xn_r[...] = xn
    y_r[...] = jax.nn.relu(_dot(xn, wy_r[...]) + by_r[...])

  g = N // blk
  dspec = [
      pl.BlockSpec((blk, H), lambda i: (i, 0)),
      pl.BlockSpec((2, blk, H), lambda i: (0, i, 0)),
      pl.BlockSpec((blk, 16), lambda i: (i, 0)),
      pl.BlockSpec((blk, 1), lambda i: (i, 0)),
      pl.BlockSpec((2, blk, H), lambda i: (0, i, 0)),
      pl.BlockSpec((blk, 1), lambda i: (i, 0)),
  ]
  wspec = [_full(w.shape) for w in wts]
  return pl.pallas_call(
      body,
      grid=(g,),
      in_specs=dspec + wspec,
      out_specs=[pl.BlockSpec((blk, H), lambda i: (i, 0))] * 2,
      out_shape=[jax.ShapeDtypeStruct((N, H), F32)] * 2,
  )(x, accx, histn, rdega, accf, rdegf, *wts)


def _tc_frag_layer(xf, acc_a2f, acc_f2f, rca, rcf, wts):
  def body(xf_r, aa_r, af_r, rca_r, rcf_r, wa1_r, ba1_r, wa2_r, ba2_r,
           wff1_r, bff1_r, wff2_r, bff2_r, wcf_r, wca_r, bcf_r, o_r):
    a2f_m = (aa_r[0] + aa_r[1]) * rca_r[...]
    f2f_m = (af_r[0] + af_r[1]) * rcf_r[...]
    a2f = jax.nn.relu(
        _dot(jax.nn.relu(_dot(a2f_m, wa1_r[...]) + ba1_r[...]), wa2_r[...])
        + ba2_r[...])
    f2f = jax.nn.relu(
        _dot(jax.nn.relu(_dot(f2f_m, wff1_r[...]) + bff1_r[...]), wff2_r[...])
        + bff2_r[...])
    o_r[...] = xf_r[...] + jax.nn.relu(
        _dot(f2f, wcf_r[...]) + _dot(a2f, wca_r[...]) + bcf_r[...])

  specs = ([_full((NF, H)), _full((2, NF, H)), _full((2, NF, H)),
            _full((NF, 1)), _full((NF, 1))]
           + [_full(w.shape) for w in wts])
  return pl.pallas_call(
      body,
      in_specs=specs,
      out_specs=_full((NF, H)),
      out_shape=jax.ShapeDtypeStruct((NF, H), F32),
  )(xf, acc_a2f, acc_f2f, rca, rcf, *wts)


def _tc_seg(x, b2d, n, blk):
  def body(x_r, b_r, s_r, c_r):
    i = pl.program_id(0)

    @pl.when(i == 0)
    def _():
      s_r[...] = jnp.zeros_like(s_r)
      c_r[...] = jnp.zeros_like(c_r)

    io = lax.broadcasted_iota(jnp.int32, (1, BSZ), 1).astype(F32)
    oh = (b_r[...] == io).astype(F32)
    s_r[...] += lax.dot_general(oh, x_r[...], (((0,), (0,)), ((), ())),
                                preferred_element_type=F32,
                                precision=lax.Precision.HIGHEST)
    c_r[...] += lax.dot_general(oh, jnp.ones((blk, 1), F32),
                                (((0,), (0,)), ((), ())),
                                preferred_element_type=F32,
                                precision=lax.Precision.HIGHEST)

  return pl.pallas_call(
      body,
      grid=(n // blk,),
      in_specs=[pl.BlockSpec((blk, H), lambda i: (i, 0)),
                pl.BlockSpec((blk, 1), lambda i: (i, 0))],
      out_specs=[_full((BSZ, H)), _full((BSZ, 1))],
      out_shape=[jax.ShapeDtypeStruct((BSZ, H), F32),
                 jax.ShapeDtypeStruct((BSZ, 1), F32)],
  )(x, b2d)


def _tc_final(sx, cx, sf, cf, wts):
  def body(sx_r, cx_r, sf_r, cf_r, wa1_r, ba1_r, wa2_r, ba2_r,
           wf1_r, bf1_r, wf2_r, bf2_r, wo_r, bo_r, o_r):
    mx = sx_r[...] * (1.0 / jnp.maximum(cx_r[...], 1.0))
    mf = sf_r[...] * (1.0 / jnp.maximum(cf_r[...], 1.0))
    xg = jax.nn.relu(
        _dot(jax.nn.relu(_dot(mx, wa1_r[...]) + ba1_r[...]), wa2_r[...])
        + ba2_r[...])
    xf = jax.nn.relu(
        _dot(jax.nn.relu(_dot(mf, wf1_r[...]) + bf1_r[...]), wf2_r[...])
        + bf2_r[...])
    o_r[...] = _dot(xg + xf, wo_r[...]) + bo_r[...]

  specs = ([_full((BSZ, H)), _full((BSZ, 1)), _full((BSZ, H)), _full((BSZ, 1))]
           + [_full(w.shape) for w in wts])
  return pl.pallas_call(
      body,
      in_specs=specs,
      out_specs=_full((BSZ, 1)),
      out_shape=jax.ShapeDtypeStruct((BSZ, 1), F32),
  )(sx, cx, sf, cf, *wts)


def kernel(params, x_atoms, edge_index, edge_attr, fragment_types,
           frag_row, frag_col, higher_edge_index, batch, fragments_batch):
  row_e, col_e = edge_index[0], edge_index[1]
  he0, he1 = higher_edge_index[0], higher_edge_index[1]
  iota_n = jnp.arange(N, dtype=jnp.int32)
  iota_nf = jnp.arange(NF, dtype=jnp.int32)

  tabF = jnp.eye(2, H, dtype=F32)
  gF = jnp.concatenate([jnp.zeros((FE,), jnp.int32), jnp.ones((HE,), jnp.int32)])
  sF = jnp.concatenate([frag_col, he1])
  tabN = jnp.eye(17, H, dtype=F32)
  gN = jnp.concatenate([edge_attr.astype(jnp.int32),
                        jnp.full((FE,), 16, jnp.int32)])
  sN = jnp.concatenate([col_e, frag_row])
  embx_p, embf_p, cntF_p, histc_p = _sc_fused([
      [(params["atom_emb"], x_atoms.astype(jnp.int32), iota_n, N),
       (params["frag_emb"], fragment_types.astype(jnp.int32), iota_nf, NF),
       (tabF, gF, sF, NF)],
      [(tabN, gN, sN, N)],
  ])
  x = _tc_add2(embx_p[:, :N, :], N)
  x_frag = _tc_add2(embf_p[:, :NF, :], NF)
  histn, rdeg_a, rdeg_f = _tc_prep_atom(histc_p[:, :N, :16],
                                        histc_p[:, :N, 16:17])
  rc_a2f, rc_f2f = _tc_prep_frag(cntF_p[:, :NF, :2])

  (accx_p,) = _sc_fused([[(x, row_e, col_e, N)]])
  acc_a2f_p = None
  for li in range(NUM_LAYERS):
    p = params["layers"][li]
    a1, a2 = p["a2a_after"]
    f1, f2 = p["f2a_after"]
    wts_atom = (
        p["bond_emb"],
        a1["w"][:H, :], a1["w"][H:, :], a1["b"].reshape(1, H),
        a2["w"], a2["b"].reshape(1, H),
        f1["w"], f1["b"].reshape(1, H), f2["w"], f2["b"].reshape(1, H),
        p["combine_atom"][0]["w"][:H, :], p["combine_atom"][0]["w"][H:, :],
        p["combine_atom"][0]["b"].reshape(1, H),
        p["a2f_before"][0]["w"], p["a2f_before"][0]["b"].reshape(1, H),
    )
    accf_p, accff_p = _sc_fused([[(x_frag, frag_col, frag_row, N),
                                  (x_frag, he0, he1, NF)]])
    x, y = _tc_atom_layer(x, accx_p[:, :N, :], histn, rdeg_a,
                          accf_p[:, :N, :], rdeg_f, wts_atom)
    if li < NUM_LAYERS - 1:
      acc_a2f_p, accx_p = _sc_fused([[(y, frag_row, frag_col, NF),
                                      (x, row_e, col_e, N)]])
    else:
      (acc_a2f_p,) = _sc_fused([[(y, frag_row, frag_col, NF)]])

    q1, q2 = p["a2f_after"]
    r1, r2 = p["f2f_after"]
    wts_frag = (
        q1["w"], q1["b"].reshape(1, H), q2["w"], q2["b"].reshape(1, H),
        r1["w"], r1["b"].reshape(1, H), r2["w"], r2["b"].reshape(1, H),
        p["combine_frag"][0]["w"][:H, :], p["combine_frag"][0]["w"][H:, :],
        p["combine_frag"][0]["b"].reshape(1, H),
    )
    x_frag = _tc_frag_layer(x_frag, acc_a2f_p[:, :NF, :], accff_p[:, :NF, :],
                            rc_a2f, rc_f2f, wts_frag)

  sx, cx = _tc_seg(x, batch.astype(F32).reshape(N, 1), N, 2000)
  sf, cf = _tc_seg(x_frag, fragments_batch.astype(F32).reshape(NF, 1), NF, NF)
  ao1, ao2 = params["atom_out"]
  fo1, fo2 = params["frag_out"]
  wts_fin = (ao1["w"], ao1["b"].reshape(1, H), ao2["w"], ao2["b"].reshape(1, H),
             fo1["w"], fo1["b"].reshape(1, H), fo2["w"], fo2["b"].reshape(1, H),
             params["out"][0]["w"], params["out"][0]["b"].reshape(1, 1))
  return _tc_final(sx, cx, sf, cf, wts_fin)


# speedup vs baseline: 1.0472x; 1.0472x over previous
import functools

import jax
import jax.numpy as jnp
from jax import lax
from jax.experimental import pallas as pl
from jax.experimental.pallas import tpu as pltpu
from jax.experimental.pallas import tpu_sc as plsc

H = 128
N = 10000
E = 320000
NF = 2000
FE = 10000
HE = 8000
NUM_LAYERS = 3
BSZ = 128

NC, NS = 2, 16
NW = NC * NS
CHUNK = 128

F32 = jnp.float32


def _round_up(a, m):
  return (a + m - 1) // m * m


def _sc_fused(rounds):
  specs = []
  acc_rows = 0
  for rnd in rounds:
    off = 0
    for (table, gidx, sidx, n_out) in rnd:
      e0 = gidx.shape[0]
      nch = -(-e0 // (NW * CHUNK))
      epad = nch * NW * CHUNK
      n_pad = _round_up(n_out + 8, NS * 8)
      sidx = sidx + off
      if epad != e0:
        pad = epad - e0
        gidx = jnp.concatenate([gidx, jnp.zeros((pad,), jnp.int32)])
        sidx = jnp.concatenate([sidx, jnp.full((pad,), off + n_out,
                                                jnp.int32)])
      specs.append(dict(
          table=table,
          g3=gidx.reshape(NW, nch, CHUNK),
          s3=sidx.reshape(NW, nch, CHUNK),
          nch=nch, n_pad=n_pad, off=off, rz=n_pad // NS))
      off += n_pad
    acc_rows = max(acc_rows, off)
  zeros = jnp.zeros((acc_rows, H), F32)
  mesh = plsc.VectorSubcoreMesh(core_axis_name="c", subcore_axis_name="s",
                                num_cores=NC, num_subcores=NS)
  n_ops_per_round = [len(r) for r in rounds]

  def body(*refs):
    k = 0
    ins = []
    for _ in specs:
      ins.append(refs[k:k + 3])
      k += 3
    z_h = refs[k]
    outs = refs[k + 1:k + 1 + len(specs)]
    gi_v, si_v, rows_v, acc_s, sem = refs[k + 1 + len(specs):]

    cid = lax.axis_index("c")
    sid = lax.axis_index("s")
    wid = cid * NS + sid

    op_i = 0
    for r, n_ops in enumerate(n_ops_per_round):
      rops = list(range(op_i, op_i + n_ops))
      op_i += n_ops
      for oi in rops:
        sp = specs[oi]
        dst = sp["off"] + sid * sp["rz"]
        pltpu.sync_copy(z_h.at[pl.ds(dst, sp["rz"])],
                        acc_s.at[pl.ds(dst, sp["rz"])])
      plsc.subcore_barrier()
      for oi in rops:
        sp = specs[oi]
        table_h, g_h, s_h = ins[oi]

        def step(i, carry, table_h=table_h, g_h=g_h, s_h=s_h):
          pltpu.sync_copy(g_h.at[wid, i], gi_v)
          pltpu.async_copy(table_h.at[gi_v], rows_v, sem).wait()
          pltpu.sync_copy(s_h.at[wid, i], si_v)
          pltpu.sync_copy(rows_v, acc_s.at[si_v], add=True)
          return carry

        lax.fori_loop(0, sp["nch"], step, 0)
      plsc.subcore_barrier()
      for oi in rops:
        sp = specs[oi]
        src = sp["off"] + sid * sp["rz"]
        pltpu.sync_copy(acc_s.at[pl.ds(src, sp["rz"])],
                        outs[oi].at[cid, pl.ds(sid * sp["rz"], sp["rz"])])
      plsc.subcore_barrier()

  call = pl.kernel(
      body,
      out_type=[jax.ShapeDtypeStruct((NC, sp["n_pad"], H), F32)
                for sp in specs],
      mesh=mesh,
      scratch_types=[
          pltpu.VMEM((CHUNK,), jnp.int32),
          pltpu.VMEM((CHUNK,), jnp.int32),
          pltpu.VMEM((CHUNK, H), F32),
          pltpu.VMEM_SHARED((acc_rows, H), F32),
          pltpu.SemaphoreType.DMA,
      ],
  )
  args = []
  for sp in specs:
    args += [sp["table"], sp["g3"], sp["s3"]]
  args.append(zeros)
  res = call(*args)
  return list(res) if isinstance(res, (list, tuple)) else [res]


def _dot(a, b):
  return jnp.dot(a, b, preferred_element_type=F32)


def _full(shape):
  return pl.BlockSpec(shape, lambda *i: (0,) * len(shape))


def _tc_add2(p2, n, blk=2000):
  def body(p_r, o_r):
    o_r[...] = p_r[0] + p_r[1]

  return pl.pallas_call(
      body,
      grid=(n // blk,),
      in_specs=[pl.BlockSpec((2, blk, H), lambda i: (0, i, 0))],
      out_specs=pl.BlockSpec((blk, H), lambda i: (i, 0)),
      out_shape=jax.ShapeDtypeStruct((n, H), F32),
  )(p2)


def _tc_prep_atom(hist2, cnt2):
  def body(h_r, c_r, hn_r, ra_r, rf_r):
    hist = h_r[0] + h_r[1]
    deg = jnp.sum(hist, axis=1, keepdims=True)
    ra = 1.0 / jnp.maximum(deg, 1.0)
    hn_r[...] = hist * ra
    ra_r[...] = ra
    rf_r[...] = 1.0 / jnp.maximum(c_r[0] + c_r[1], 1.0)

  return pl.pallas_call(
      body,
      in_specs=[_full((2, N, 16)), _full((2, N, 1))],
      out_specs=[_full((N, 16)), _full((N, 1)), _full((N, 1))],
      out_shape=[
          jax.ShapeDtypeStruct((N, 16), F32),
          jax.ShapeDtypeStruct((N, 1), F32),
          jax.ShapeDtypeStruct((N, 1), F32),
      ],
  )(hist2, cnt2)


def _tc_prep_frag(c2):
  def body(c_r, ra_r, rf_r):
    c = c_r[0] + c_r[1]
    r = 1.0 / jnp.maximum(c, 1.0)
    ra_r[...] = r[:, 0:1]
    rf_r[...] = r[:, 1:2]

  return pl.pallas_call(
      body,
      in_specs=[_full((2, NF, 2))],
      out_specs=[_full((NF, 1)), _full((NF, 1))],
      out_shape=[
          jax.ShapeDtypeStruct((NF, 1), F32),
          jax.ShapeDtypeStruct((NF, 1), F32),
      ],
  )(c2)


def _tc_atom_layer(x, accx, histn, rdega, accf, rdegf, wts, blk=2000):
  def body(x_r, ax_r, hn_r, ra_r, af_r, rf_r, be_r, w1x_r, w1e_r, b1_r,
           w2_r, b2_r, wf1_r, bf1_r, wf2_r, bf2_r, wca_r, wcf_r, bc_r,
           wy_r, by_r, xn_r, y_r):
    xm = (ax_r[0] + ax_r[1]) * ra_r[...]
    em = _dot(hn_r[...], be_r[...])
    h = jax.nn.relu(_dot(xm, w1x_r[...]) + _dot(em, w1e_r[...]) + b1_r[...])
    a2a = jax.nn.relu(_dot(h, w2_r[...]) + b2_r[...])
    fm = (af_r[0] + af_r[1]) * rf_r[...]
    f2a = jax.nn.relu(
        _dot(jax.nn.relu(_dot(fm, wf1_r[...]) + bf1_r[...]), wf2_r[...])
        + bf2_r[...])
    xn = x_r[...] + jax.nn.relu(
        _dot(a2a, wca_r[...]) + _dot(f2a, wcf_r[...]) + bc_r[...])
    xn_r[...] = xn
    y_r[...] = jax.nn.relu(_dot(xn, wy_r[...]) + by_r[...])

  g = N // blk
  dspec = [
      pl.BlockSpec((blk, H), lambda i: (i, 0)),
      pl.BlockSpec((2, blk, H), lambda i: (0, i, 0)),
      pl.BlockSpec((blk, 16), lambda i: (i, 0)),
      pl.BlockSpec((blk, 1), lambda i: (i, 0)),
      pl.BlockSpec((2, blk, H), lambda i: (0, i, 0)),
      pl.BlockSpec((blk, 1), lambda i: (i, 0)),
  ]
  wspec = [_full(w.shape) for w in wts]
  return pl.pallas_call(
      body,
      grid=(g,),
      in_specs=dspec + wspec,
      out_specs=[pl.BlockSpec((blk, H), lambda i: (i, 0))] * 2,
      out_shape=[jax.ShapeDtypeStruct((N, H), F32)] * 2,
  )(x, accx, histn, rdega, accf, rdegf, *wts)


def _tc_frag_layer(xf, acc_a2f, acc_f2f, rca, rcf, wts):
  def body(xf_r, aa_r, af_r, rca_r, rcf_r, wa1_r, ba1_r, wa2_r, ba2_r,
           wff1_r, bff1_r, wff2_r, bff2_r, wcf_r, wca_r, bcf_r, o_r):
    a2f_m = (aa_r[0] + aa_r[1]) * rca_r[...]
    f2f_m = (af_r[0] + af_r[1]) * rcf_r[...]
    a2f = jax.nn.relu(
        _dot(jax.nn.relu(_dot(a2f_m, wa1_r[...]) + ba1_r[...]), wa2_r[...])
        + ba2_r[...])
    f2f = jax.nn.relu(
        _dot(jax.nn.relu(_dot(f2f_m, wff1_r[...]) + bff1_r[...]), wff2_r[...])
        + bff2_r[...])
    o_r[...] = xf_r[...] + jax.nn.relu(
        _dot(f2f, wcf_r[...]) + _dot(a2f, wca_r[...]) + bcf_r[...])

  specs = ([_full((NF, H)), _full((2, NF, H)), _full((2, NF, H)),
            _full((NF, 1)), _full((NF, 1))]
           + [_full(w.shape) for w in wts])
  return pl.pallas_call(
      body,
      in_specs=specs,
      out_specs=_full((NF, H)),
      out_shape=jax.ShapeDtypeStruct((NF, H), F32),
  )(xf, acc_a2f, acc_f2f, rca, rcf, *wts)


def _tc_seg(x, b2d, n, blk):
  def body(x_r, b_r, s_r, c_r):
    i = pl.program_id(0)

    @pl.when(i == 0)
    def _():
      s_r[...] = jnp.zeros_like(s_r)
      c_r[...] = jnp.zeros_like(c_r)

    io = lax.broadcasted_iota(jnp.int32, (1, BSZ), 1).astype(F32)
    oh = (b_r[...] == io).astype(F32)
    s_r[...] += lax.dot_general(oh, x_r[...], (((0,), (0,)), ((), ())),
                                preferred_element_type=F32)
    c_r[...] += lax.dot_general(oh, jnp.ones((blk, 1), F32),
                                (((0,), (0,)), ((), ())),
                                preferred_element_type=F32)

  return pl.pallas_call(
      body,
      grid=(n // blk,),
      in_specs=[pl.BlockSpec((blk, H), lambda i: (i, 0)),
                pl.BlockSpec((blk, 1), lambda i: (i, 0))],
      out_specs=[_full((BSZ, H)), _full((BSZ, 1))],
      out_shape=[jax.ShapeDtypeStruct((BSZ, H), F32),
                 jax.ShapeDtypeStruct((BSZ, 1), F32)],
  )(x, b2d)


def _tc_final(sx, cx, sf, cf, wts):
  def body(sx_r, cx_r, sf_r, cf_r, wa1_r, ba1_r, wa2_r, ba2_r,
           wf1_r, bf1_r, wf2_r, bf2_r, wo_r, bo_r, o_r):
    mx = sx_r[...] * (1.0 / jnp.maximum(cx_r[...], 1.0))
    mf = sf_r[...] * (1.0 / jnp.maximum(cf_r[...], 1.0))
    xg = jax.nn.relu(
        _dot(jax.nn.relu(_dot(mx, wa1_r[...]) + ba1_r[...]), wa2_r[...])
        + ba2_r[...])
    xf = jax.nn.relu(
        _dot(jax.nn.relu(_dot(mf, wf1_r[...]) + bf1_r[...]), wf2_r[...])
        + bf2_r[...])
    o_r[...] = _dot(xg + xf, wo_r[...]) + bo_r[...]

  specs = ([_full((BSZ, H)), _full((BSZ, 1)), _full((BSZ, H)), _full((BSZ, 1))]
           + [_full(w.shape) for w in wts])
  return pl.pallas_call(
      body,
      in_specs=specs,
      out_specs=_full((BSZ, 1)),
      out_shape=jax.ShapeDtypeStruct((BSZ, 1), F32),
  )(sx, cx, sf, cf, *wts)


def kernel(params, x_atoms, edge_index, edge_attr, fragment_types,
           frag_row, frag_col, higher_edge_index, batch, fragments_batch):
  row_e, col_e = edge_index[0], edge_index[1]
  he0, he1 = higher_edge_index[0], higher_edge_index[1]
  iota_n = jnp.arange(N, dtype=jnp.int32)
  iota_nf = jnp.arange(NF, dtype=jnp.int32)

  tabF = jnp.eye(2, H, dtype=F32)
  gF = jnp.concatenate([jnp.zeros((FE,), jnp.int32), jnp.ones((HE,), jnp.int32)])
  sF = jnp.concatenate([frag_col, he1])
  tabN = jnp.eye(17, H, dtype=F32)
  gN = jnp.concatenate([edge_attr.astype(jnp.int32),
                        jnp.full((FE,), 16, jnp.int32)])
  sN = jnp.concatenate([col_e, frag_row])
  embx_p, embf_p, cntF_p, histc_p = _sc_fused([
      [(params["atom_emb"], x_atoms.astype(jnp.int32), iota_n, N),
       (params["frag_emb"], fragment_types.astype(jnp.int32), iota_nf, NF),
       (tabF, gF, sF, NF)],
      [(tabN, gN, sN, N)],
  ])
  x = _tc_add2(embx_p[:, :N, :], N)
  x_frag = _tc_add2(embf_p[:, :NF, :], NF)
  histn, rdeg_a, rdeg_f = _tc_prep_atom(histc_p[:, :N, :16],
                                        histc_p[:, :N, 16:17])
  rc_a2f, rc_f2f = _tc_prep_frag(cntF_p[:, :NF, :2])

  (accx_p,) = _sc_fused([[(x, row_e, col_e, N)]])
  acc_a2f_p = None
  for li in range(NUM_LAYERS):
    p = params["layers"][li]
    a1, a2 = p["a2a_after"]
    f1, f2 = p["f2a_after"]
    wts_atom = (
        p["bond_emb"],
        a1["w"][:H, :], a1["w"][H:, :], a1["b"].reshape(1, H),
        a2["w"], a2["b"].reshape(1, H),
        f1["w"], f1["b"].reshape(1, H), f2["w"], f2["b"].reshape(1, H),
        p["combine_atom"][0]["w"][:H, :], p["combine_atom"][0]["w"][H:, :],
        p["combine_atom"][0]["b"].reshape(1, H),
        p["a2f_before"][0]["w"], p["a2f_before"][0]["b"].reshape(1, H),
    )
    accf_p, accff_p = _sc_fused([[(x_frag, frag_col, frag_row, N),
                                  (x_frag, he0, he1, NF)]])
    x, y = _tc_atom_layer(x, accx_p[:, :N, :], histn, rdeg_a,
                          accf_p[:, :N, :], rdeg_f, wts_atom)
    if li < NUM_LAYERS - 1:
      acc_a2f_p, accx_p = _sc_fused([[(y, frag_row, frag_col, NF),
                                      (x, row_e, col_e, N)]])
    else:
      (acc_a2f_p,) = _sc_fused([[(y, frag_row, frag_col, NF)]])

    q1, q2 = p["a2f_after"]
    r1, r2 = p["f2f_after"]
    wts_frag = (
        q1["w"], q1["b"].reshape(1, H), q2["w"], q2["b"].reshape(1, H),
        r1["w"], r1["b"].reshape(1, H), r2["w"], r2["b"].reshape(1, H),
        p["combine_frag"][0]["w"][:H, :], p["combine_frag"][0]["w"][H:, :],
        p["combine_frag"][0]["b"].reshape(1, H),
    )
    x_frag = _tc_frag_layer(x_frag, acc_a2f_p[:, :NF, :], accff_p[:, :NF, :],
                            rc_a2f, rc_f2f, wts_frag)

  sx, cx = _tc_seg(x, batch.astype(F32).reshape(N, 1), N, 2000)
  sf, cf = _tc_seg(x_frag, fragments_batch.astype(F32).reshape(NF, 1), NF, NF)
  ao1, ao2 = params["atom_out"]
  fo1, fo2 = params["frag_out"]
  wts_fin = (ao1["w"], ao1["b"].reshape(1, H), ao2["w"], ao2["b"].reshape(1, H),
             fo1["w"], fo1["b"].reshape(1, H), fo2["w"], fo2["b"].reshape(1, H),
             params["out"][0]["w"], params["out"][0]["b"].reshape(1, 1))
  return _tc_final(sx, cx, sf, cf, wts_fin)


# speedup vs baseline: 1.4128x; 1.3492x over previous
import functools

import jax
import jax.numpy as jnp
from jax import lax
from jax.experimental import pallas as pl
from jax.experimental.pallas import tpu as pltpu
from jax.experimental.pallas import tpu_sc as plsc

H = 128
N = 10000
E = 320000
NF = 2000
FE = 10000
HE = 8000
NUM_LAYERS = 3
BSZ = 128

NC, NS = 2, 16
NW = NC * NS
CHUNK = 128

F32 = jnp.float32


def _round_up(a, m):
  return (a + m - 1) // m * m


def _sc_fused(rounds):
  specs = []
  acc_rows = 0
  for rnd in rounds:
    off = 0
    for (table, gidx, sidx, n_out, mode) in rnd:
      e0 = sidx.shape[0]
      nch = -(-e0 // (NW * CHUNK))
      epad = nch * NW * CHUNK
      n_pad = _round_up(n_out + 8, NS * 8)
      sidx = sidx + off
      if epad != e0:
        pad = epad - e0
        sidx = jnp.concatenate([sidx, jnp.full((pad,), off + n_out,
                                                jnp.int32)])
        if mode != "const":
          gidx = jnp.concatenate([gidx, jnp.zeros((pad,), jnp.int32)])
      if mode == "const":
        g3 = None
      else:
        g3 = gidx.reshape(NW, nch, CHUNK)
        if mode == "replicate":
          t_rows = table.shape[0]
          table = jnp.tile(table, (NW, 1))
          g3 = g3 + (jnp.arange(NW, dtype=jnp.int32) * t_rows)[:, None, None]
      specs.append(dict(
          table=table, g3=g3, s3=sidx.reshape(NW, nch, CHUNK), mode=mode,
          nch=nch, n_pad=n_pad, off=off, rz=n_pad // NS))
      off += n_pad
    acc_rows = max(acc_rows, off)
  zeros = jnp.zeros((acc_rows, H), F32)
  mesh = plsc.VectorSubcoreMesh(core_axis_name="c", subcore_axis_name="s",
                                num_cores=NC, num_subcores=NS)
  n_ops_per_round = [len(r) for r in rounds]

  def body(*refs):
    k = 0
    ins = []
    for sp in specs:
      nin = 2 if sp["mode"] == "const" else 3
      ins.append(refs[k:k + nin])
      k += nin
    z_h = refs[k]
    outs = refs[k + 1:k + 1 + len(specs)]
    gi_v, si_v, rows_v, acc_s, sem = refs[k + 1 + len(specs):]

    cid = lax.axis_index("c")
    sid = lax.axis_index("s")
    wid = cid * NS + sid

    op_i = 0
    for r, n_ops in enumerate(n_ops_per_round):
      rops = list(range(op_i, op_i + n_ops))
      op_i += n_ops
      for oi in rops:
        sp = specs[oi]
        dst = sp["off"] + sid * sp["rz"]
        pltpu.sync_copy(z_h.at[pl.ds(dst, sp["rz"])],
                        acc_s.at[pl.ds(dst, sp["rz"])])
      plsc.subcore_barrier()
      for oi in rops:
        sp = specs[oi]
        if sp["mode"] == "const":
          table_h, s_h = ins[oi]
          pltpu.sync_copy(table_h, rows_v)

          def step(i, carry, s_h=s_h):
            pltpu.sync_copy(s_h.at[wid, i], si_v)
            pltpu.sync_copy(rows_v, acc_s.at[si_v], add=True)
            return carry
        else:
          table_h, g_h, s_h = ins[oi]

          def step(i, carry, table_h=table_h, g_h=g_h, s_h=s_h):
            pltpu.sync_copy(g_h.at[wid, i], gi_v)
            pltpu.async_copy(table_h.at[gi_v], rows_v, sem).wait()
            pltpu.sync_copy(s_h.at[wid, i], si_v)
            pltpu.sync_copy(rows_v, acc_s.at[si_v], add=True)
            return carry

        lax.fori_loop(0, sp["nch"], step, 0)
      plsc.subcore_barrier()
      for oi in rops:
        sp = specs[oi]
        src = sp["off"] + sid * sp["rz"]
        pltpu.sync_copy(acc_s.at[pl.ds(src, sp["rz"])],
                        outs[oi].at[cid, pl.ds(sid * sp["rz"], sp["rz"])])
      plsc.subcore_barrier()

  call = pl.kernel(
      body,
      out_type=[jax.ShapeDtypeStruct((NC, sp["n_pad"], H), F32)
                for sp in specs],
      mesh=mesh,
      scratch_types=[
          pltpu.VMEM((CHUNK,), jnp.int32),
          pltpu.VMEM((CHUNK,), jnp.int32),
          pltpu.VMEM((CHUNK, H), F32),
          pltpu.VMEM_SHARED((acc_rows, H), F32),
          pltpu.SemaphoreType.DMA,
      ],
  )
  args = []
  for sp in specs:
    if sp["mode"] == "const":
      args += [sp["table"], sp["s3"]]
    else:
      args += [sp["table"], sp["g3"], sp["s3"]]
  args.append(zeros)
  res = call(*args)
  return list(res) if isinstance(res, (list, tuple)) else [res]


def _dot(a, b):
  return jnp.dot(a, b, preferred_element_type=F32)


def _full(shape):
  return pl.BlockSpec(shape, lambda *i: (0,) * len(shape))


def _tc_add2(p2, n, blk=2000):
  def body(p_r, o_r):
    o_r[...] = p_r[0] + p_r[1]

  return pl.pallas_call(
      body,
      grid=(n // blk,),
      in_specs=[pl.BlockSpec((2, blk, H), lambda i: (0, i, 0))],
      out_specs=pl.BlockSpec((blk, H), lambda i: (i, 0)),
      out_shape=jax.ShapeDtypeStruct((n, H), F32),
  )(p2)


def _tc_prep_atom(hist2, cnt2):
  def body(h_r, c_r, hn_r, ra_r, rf_r):
    hist = h_r[0] + h_r[1]
    deg = jnp.sum(hist, axis=1, keepdims=True)
    ra = 1.0 / jnp.maximum(deg, 1.0)
    hn_r[...] = hist * ra
    ra_r[...] = ra
    rf_r[...] = 1.0 / jnp.maximum(c_r[0] + c_r[1], 1.0)

  return pl.pallas_call(
      body,
      in_specs=[_full((2, N, 16)), _full((2, N, 1))],
      out_specs=[_full((N, 16)), _full((N, 1)), _full((N, 1))],
      out_shape=[
          jax.ShapeDtypeStruct((N, 16), F32),
          jax.ShapeDtypeStruct((N, 1), F32),
          jax.ShapeDtypeStruct((N, 1), F32),
      ],
  )(hist2, cnt2)


def _tc_prep_frag(ca2, cf2):
  def body(ca_r, cf_r, ra_r, rf_r):
    ra_r[...] = 1.0 / jnp.maximum(ca_r[0] + ca_r[1], 1.0)
    rf_r[...] = 1.0 / jnp.maximum(cf_r[0] + cf_r[1], 1.0)

  return pl.pallas_call(
      body,
      in_specs=[_full((2, NF, 1)), _full((2, NF, 1))],
      out_specs=[_full((NF, 1)), _full((NF, 1))],
      out_shape=[
          jax.ShapeDtypeStruct((NF, 1), F32),
          jax.ShapeDtypeStruct((NF, 1), F32),
      ],
  )(ca2, cf2)


def _tc_atom_layer(x, accx, histn, rdega, accf, rdegf, wts, blk=2000):
  def body(x_r, ax_r, hn_r, ra_r, af_r, rf_r, be_r, w1x_r, w1e_r, b1_r,
           w2_r, b2_r, wf1_r, bf1_r, wf2_r, bf2_r, wca_r, wcf_r, bc_r,
           wy_r, by_r, xn_r, y_r):
    xm = (ax_r[0] + ax_r[1]) * ra_r[...]
    em = _dot(hn_r[...], be_r[...])
    h = jax.nn.relu(_dot(xm, w1x_r[...]) + _dot(em, w1e_r[...]) + b1_r[...])
    a2a = jax.nn.relu(_dot(h, w2_r[...]) + b2_r[...])
    fm = (af_r[0] + af_r[1]) * rf_r[...]
    f2a = jax.nn.relu(
        _dot(jax.nn.relu(_dot(fm, wf1_r[...]) + bf1_r[...]), wf2_r[...])
        + bf2_r[...])
    xn = x_r[...] + jax.nn.relu(
        _dot(a2a, wca_r[...]) + _dot(f2a, wcf_r[...]) + bc_r[...])
    xn_r[...] = xn
    y_r[...] = jax.nn.relu(_dot(xn, wy_r[...]) + by_r[...])

  g = N // blk
  dspec = [
      pl.BlockSpec((blk, H), lambda i: (i, 0)),
      pl.BlockSpec((2, blk, H), lambda i: (0, i, 0)),
      pl.BlockSpec((blk, 16), lambda i: (i, 0)),
      pl.BlockSpec((blk, 1), lambda i: (i, 0)),
      pl.BlockSpec((2, blk, H), lambda i: (0, i, 0)),
      pl.BlockSpec((blk, 1), lambda i: (i, 0)),
  ]
  wspec = [_full(w.shape) for w in wts]
  return pl.pallas_call(
      body,
      grid=(g,),
      in_specs=dspec + wspec,
      out_specs=[pl.BlockSpec((blk, H), lambda i: (i, 0))] * 2,
      out_shape=[jax.ShapeDtypeStruct((N, H), F32)] * 2,
  )(x, accx, histn, rdega, accf, rdegf, *wts)


def _tc_frag_layer(xf, acc_a2f, acc_f2f, rca, rcf, wts):
  def body(xf_r, aa_r, af_r, rca_r, rcf_r, wa1_r, ba1_r, wa2_r, ba2_r,
           wff1_r, bff1_r, wff2_r, bff2_r, wcf_r, wca_r, bcf_r, o_r):
    a2f_m = (aa_r[0] + aa_r[1]) * rca_r[...]
    f2f_m = (af_r[0] + af_r[1]) * rcf_r[...]
    a2f = jax.nn.relu(
        _dot(jax.nn.relu(_dot(a2f_m, wa1_r[...]) + ba1_r[...]), wa2_r[...])
        + ba2_r[...])
    f2f = jax.nn.relu(
        _dot(jax.nn.relu(_dot(f2f_m, wff1_r[...]) + bff1_r[...]), wff2_r[...])
        + bff2_r[...])
    o_r[...] = xf_r[...] + jax.nn.relu(
        _dot(f2f, wcf_r[...]) + _dot(a2f, wca_r[...]) + bcf_r[...])

  specs = ([_full((NF, H)), _full((2, NF, H)), _full((2, NF, H)),
            _full((NF, 1)), _full((NF, 1))]
           + [_full(w.shape) for w in wts])
  return pl.pallas_call(
      body,
      in_specs=specs,
      out_specs=_full((NF, H)),
      out_shape=jax.ShapeDtypeStruct((NF, H), F32),
  )(xf, acc_a2f, acc_f2f, rca, rcf, *wts)


def _tc_seg(x, b2d, n, blk):
  def body(x_r, b_r, s_r, c_r):
    i = pl.program_id(0)

    @pl.when(i == 0)
    def _():
      s_r[...] = jnp.zeros_like(s_r)
      c_r[...] = jnp.zeros_like(c_r)

    io = lax.broadcasted_iota(jnp.int32, (1, BSZ), 1).astype(F32)
    oh = (b_r[...] == io).astype(F32)
    s_r[...] += lax.dot_general(oh, x_r[...], (((0,), (0,)), ((), ())),
                                preferred_element_type=F32)
    c_r[...] += lax.dot_general(oh, jnp.ones((blk, 1), F32),
                                (((0,), (0,)), ((), ())),
                                preferred_element_type=F32)

  return pl.pallas_call(
      body,
      grid=(n // blk,),
      in_specs=[pl.BlockSpec((blk, H), lambda i: (i, 0)),
                pl.BlockSpec((blk, 1), lambda i: (i, 0))],
      out_specs=[_full((BSZ, H)), _full((BSZ, 1))],
      out_shape=[jax.ShapeDtypeStruct((BSZ, H), F32),
                 jax.ShapeDtypeStruct((BSZ, 1), F32)],
  )(x, b2d)


def _tc_final(sx, cx, sf, cf, wts):
  def body(sx_r, cx_r, sf_r, cf_r, wa1_r, ba1_r, wa2_r, ba2_r,
           wf1_r, bf1_r, wf2_r, bf2_r, wo_r, bo_r, o_r):
    mx = sx_r[...] * (1.0 / jnp.maximum(cx_r[...], 1.0))
    mf = sf_r[...] * (1.0 / jnp.maximum(cf_r[...], 1.0))
    xg = jax.nn.relu(
        _dot(jax.nn.relu(_dot(mx, wa1_r[...]) + ba1_r[...]), wa2_r[...])
        + ba2_r[...])
    xf = jax.nn.relu(
        _dot(jax.nn.relu(_dot(mf, wf1_r[...]) + bf1_r[...]), wf2_r[...])
        + bf2_r[...])
    o_r[...] = _dot(xg + xf, wo_r[...]) + bo_r[...]

  specs = ([_full((BSZ, H)), _full((BSZ, 1)), _full((BSZ, H)), _full((BSZ, 1))]
           + [_full(w.shape) for w in wts])
  return pl.pallas_call(
      body,
      in_specs=specs,
      out_specs=_full((BSZ, 1)),
      out_shape=jax.ShapeDtypeStruct((BSZ, 1), F32),
  )(sx, cx, sf, cf, *wts)


def kernel(params, x_atoms, edge_index, edge_attr, fragment_types,
           frag_row, frag_col, higher_edge_index, batch, fragments_batch):
  row_e, col_e = edge_index[0], edge_index[1]
  he0, he1 = higher_edge_index[0], higher_edge_index[1]
  iota_n = jnp.arange(N, dtype=jnp.int32)
  iota_nf = jnp.arange(NF, dtype=jnp.int32)

  erow = jnp.tile(jnp.eye(1, H, dtype=F32), (CHUNK, 1))
  hist_tab = jnp.eye(16, H, dtype=F32)
  (embx_p,) = _sc_fused([
      [(params["atom_emb"], x_atoms.astype(jnp.int32), iota_n, N,
        "replicate")]])
  x = _tc_add2(embx_p[:, :N, :], N)
  (accx_p,) = _sc_fused([[(x, row_e, col_e, N, "gather")]])
  embf_p, degf_p, cnta_p, hist_p, cntf_p = _sc_fused([
      [(params["frag_emb"], fragment_types.astype(jnp.int32), iota_nf, NF,
        "replicate"),
       (erow, None, frag_row, N, "const"),
       (erow, None, frag_col, NF, "const")],
      [(hist_tab, edge_attr.astype(jnp.int32), col_e, N, "replicate"),
       (erow, None, he1, NF, "const")],
  ])
  x_frag = _tc_add2(embf_p[:, :NF, :], NF)
  histn, rdeg_a, rdeg_f = _tc_prep_atom(hist_p[:, :N, :16],
                                        degf_p[:, :N, 0:1])
  rc_a2f, rc_f2f = _tc_prep_frag(cnta_p[:, :NF, 0:1], cntf_p[:, :NF, 0:1])

  acc_a2f_p = None
  for li in range(NUM_LAYERS):
    p = params["layers"][li]
    a1, a2 = p["a2a_after"]
    f1, f2 = p["f2a_after"]
    wts_atom = (
        p["bond_emb"],
        a1["w"][:H, :], a1["w"][H:, :], a1["b"].reshape(1, H),
        a2["w"], a2["b"].reshape(1, H),
        f1["w"], f1["b"].reshape(1, H), f2["w"], f2["b"].reshape(1, H),
        p["combine_atom"][0]["w"][:H, :], p["combine_atom"][0]["w"][H:, :],
        p["combine_atom"][0]["b"].reshape(1, H),
        p["a2f_before"][0]["w"], p["a2f_before"][0]["b"].reshape(1, H),
    )
    accf_p, accff_p = _sc_fused([[(x_frag, frag_col, frag_row, N, "gather"),
                                  (x_frag, he0, he1, NF, "gather")]])
    x, y = _tc_atom_layer(x, accx_p[:, :N, :], histn, rdeg_a,
                          accf_p[:, :N, :], rdeg_f, wts_atom)
    if li < NUM_LAYERS - 1:
      acc_a2f_p, accx_p = _sc_fused([[(y, frag_row, frag_col, NF, "gather"),
                                      (x, row_e, col_e, N, "gather")]])
    else:
      (acc_a2f_p,) = _sc_fused([[(y, frag_row, frag_col, NF, "gather")]])

    q1, q2 = p["a2f_after"]
    r1, r2 = p["f2f_after"]
    wts_frag = (
        q1["w"], q1["b"].reshape(1, H), q2["w"], q2["b"].reshape(1, H),
        r1["w"], r1["b"].reshape(1, H), r2["w"], r2["b"].reshape(1, H),
        p["combine_frag"][0]["w"][:H, :], p["combine_frag"][0]["w"][H:, :],
        p["combine_frag"][0]["b"].reshape(1, H),
    )
    x_frag = _tc_frag_layer(x_frag, acc_a2f_p[:, :NF, :], accff_p[:, :NF, :],
                            rc_a2f, rc_f2f, wts_frag)

  sx, cx = _tc_seg(x, batch.astype(F32).reshape(N, 1), N, 2000)
  sf, cf = _tc_seg(x_frag, fragments_batch.astype(F32).reshape(NF, 1), NF, NF)
  ao1, ao2 = params["atom_out"]
  fo1, fo2 = params["frag_out"]
  wts_fin = (ao1["w"], ao1["b"].reshape(1, H), ao2["w"], ao2["b"].reshape(1, H),
             fo1["w"], fo1["b"].reshape(1, H), fo2["w"], fo2["b"].reshape(1, H),
             params["out"][0]["w"], params["out"][0]["b"].reshape(1, 1))
  return _tc_final(sx, cx, sf, cf, wts_fin)


# speedup vs baseline: 2.6990x; 1.9103x over previous
import functools

import jax
import jax.numpy as jnp
from jax import lax
from jax.experimental import pallas as pl
from jax.experimental.pallas import tpu as pltpu
from jax.experimental.pallas import tpu_sc as plsc

H = 128
N = 10000
E = 320000
NF = 2000
FE = 10000
HE = 8000
NUM_LAYERS = 3
BSZ = 128

NC, NS = 2, 16
NW = NC * NS
CHUNK = 128

F32 = jnp.float32


def _round_up(a, m):
  return (a + m - 1) // m * m


def _sc_fused(rounds):
  specs = []
  acc_rows = 0
  for rnd in rounds:
    off = 0
    for (table, gidx, sidx, n_out, mode) in rnd:
      e0 = sidx.shape[0]
      nch = -(-e0 // (NW * CHUNK))
      epad = nch * NW * CHUNK
      n_pad = _round_up(n_out + 8, NS * 8)
      sidx = sidx + off
      if epad != e0:
        pad = epad - e0
        spare = jnp.arange(pad, dtype=jnp.int32) % (n_pad - n_out)
        sidx = jnp.concatenate([sidx, off + n_out + spare])
        if mode != "const":
          gpad = jnp.arange(pad, dtype=jnp.int32) % table.shape[0]
          gidx = jnp.concatenate([gidx, gpad])
      if mode == "const":
        g3 = None
      else:
        g3 = gidx.reshape(NW, nch, CHUNK)
        if mode == "replicate":
          t_rows = table.shape[0]
          table = jnp.tile(table, (NW, 1))
          g3 = g3 + (jnp.arange(NW, dtype=jnp.int32) * t_rows)[:, None, None]
      specs.append(dict(
          table=table, g3=g3, s3=sidx.reshape(NW, nch, CHUNK), mode=mode,
          nch=nch, n_pad=n_pad, off=off, rz=n_pad // NS))
      off += n_pad
    acc_rows = max(acc_rows, off)
  zeros = jnp.zeros((acc_rows, H), F32)
  mesh = plsc.VectorSubcoreMesh(core_axis_name="c", subcore_axis_name="s",
                                num_cores=NC, num_subcores=NS)
  n_ops_per_round = [len(r) for r in rounds]

  def body(*refs):
    k = 0
    ins = []
    for sp in specs:
      nin = 2 if sp["mode"] == "const" else 3
      ins.append(refs[k:k + nin])
      k += nin
    z_h = refs[k]
    outs = refs[k + 1:k + 1 + len(specs)]
    gi_v, si_v, rows_v, acc_s, sem = refs[k + 1 + len(specs):]

    cid = lax.axis_index("c")
    sid = lax.axis_index("s")
    wid = cid * NS + sid

    op_i = 0
    for r, n_ops in enumerate(n_ops_per_round):
      rops = list(range(op_i, op_i + n_ops))
      op_i += n_ops
      for oi in rops:
        sp = specs[oi]
        dst = sp["off"] + sid * sp["rz"]
        pltpu.sync_copy(z_h.at[pl.ds(dst, sp["rz"])],
                        acc_s.at[pl.ds(dst, sp["rz"])])
      plsc.subcore_barrier()
      for oi in rops:
        sp = specs[oi]
        if sp["mode"] == "const":
          table_h, s_h = ins[oi]
          pltpu.sync_copy(table_h, rows_v)

          def step(i, carry, s_h=s_h):
            pltpu.sync_copy(s_h.at[wid, i], si_v)
            pltpu.sync_copy(rows_v, acc_s.at[si_v], add=True)
            return carry
        else:
          table_h, g_h, s_h = ins[oi]

          def step(i, carry, table_h=table_h, g_h=g_h, s_h=s_h):
            pltpu.sync_copy(g_h.at[wid, i], gi_v)
            pltpu.async_copy(table_h.at[gi_v], rows_v, sem).wait()
            pltpu.sync_copy(s_h.at[wid, i], si_v)
            pltpu.sync_copy(rows_v, acc_s.at[si_v], add=True)
            return carry

        lax.fori_loop(0, sp["nch"], step, 0)
      plsc.subcore_barrier()
      for oi in rops:
        sp = specs[oi]
        src = sp["off"] + sid * sp["rz"]
        pltpu.sync_copy(acc_s.at[pl.ds(src, sp["rz"])],
                        outs[oi].at[cid, pl.ds(sid * sp["rz"], sp["rz"])])
      plsc.subcore_barrier()

  call = pl.kernel(
      body,
      out_type=[jax.ShapeDtypeStruct((NC, sp["n_pad"], H), F32)
                for sp in specs],
      mesh=mesh,
      scratch_types=[
          pltpu.VMEM((CHUNK,), jnp.int32),
          pltpu.VMEM((CHUNK,), jnp.int32),
          pltpu.VMEM((CHUNK, H), F32),
          pltpu.VMEM_SHARED((acc_rows, H), F32),
          pltpu.SemaphoreType.DMA,
      ],
  )
  args = []
  for sp in specs:
    if sp["mode"] == "const":
      args += [sp["table"], sp["s3"]]
    else:
      args += [sp["table"], sp["g3"], sp["s3"]]
  args.append(zeros)
  res = call(*args)
  return list(res) if isinstance(res, (list, tuple)) else [res]


def _dot(a, b):
  return jnp.dot(a, b, preferred_element_type=F32)


def _full(shape):
  return pl.BlockSpec(shape, lambda *i: (0,) * len(shape))


def _tc_add2(p2, n, blk=2000):
  def body(p_r, o_r):
    o_r[...] = p_r[0] + p_r[1]

  return pl.pallas_call(
      body,
      grid=(n // blk,),
      in_specs=[pl.BlockSpec((2, blk, H), lambda i: (0, i, 0))],
      out_specs=pl.BlockSpec((blk, H), lambda i: (i, 0)),
      out_shape=jax.ShapeDtypeStruct((n, H), F32),
  )(p2)


def _tc_prep_atom(hist2, cnt2):
  def body(h_r, c_r, hn_r, ra_r, rf_r):
    hist = h_r[0] + h_r[1]
    deg = jnp.sum(hist, axis=1, keepdims=True)
    ra = 1.0 / jnp.maximum(deg, 1.0)
    hn_r[...] = hist * ra
    ra_r[...] = ra
    rf_r[...] = 1.0 / jnp.maximum(c_r[0] + c_r[1], 1.0)

  return pl.pallas_call(
      body,
      in_specs=[_full((2, N, 16)), _full((2, N, 1))],
      out_specs=[_full((N, 16)), _full((N, 1)), _full((N, 1))],
      out_shape=[
          jax.ShapeDtypeStruct((N, 16), F32),
          jax.ShapeDtypeStruct((N, 1), F32),
          jax.ShapeDtypeStruct((N, 1), F32),
      ],
  )(hist2, cnt2)


def _tc_prep_frag(ca2, cf2):
  def body(ca_r, cf_r, ra_r, rf_r):
    ra_r[...] = 1.0 / jnp.maximum(ca_r[0] + ca_r[1], 1.0)
    rf_r[...] = 1.0 / jnp.maximum(cf_r[0] + cf_r[1], 1.0)

  return pl.pallas_call(
      body,
      in_specs=[_full((2, NF, 1)), _full((2, NF, 1))],
      out_specs=[_full((NF, 1)), _full((NF, 1))],
      out_shape=[
          jax.ShapeDtypeStruct((NF, 1), F32),
          jax.ShapeDtypeStruct((NF, 1), F32),
      ],
  )(ca2, cf2)


def _tc_atom_layer(x, accx, histn, rdega, accf, rdegf, wts, blk=2000):
  def body(x_r, ax_r, hn_r, ra_r, af_r, rf_r, be_r, w1x_r, w1e_r, b1_r,
           w2_r, b2_r, wf1_r, bf1_r, wf2_r, bf2_r, wca_r, wcf_r, bc_r,
           wy_r, by_r, xn_r, y_r):
    xm = (ax_r[0] + ax_r[1]) * ra_r[...]
    em = _dot(hn_r[...], be_r[...])
    h = jax.nn.relu(_dot(xm, w1x_r[...]) + _dot(em, w1e_r[...]) + b1_r[...])
    a2a = jax.nn.relu(_dot(h, w2_r[...]) + b2_r[...])
    fm = (af_r[0] + af_r[1]) * rf_r[...]
    f2a = jax.nn.relu(
        _dot(jax.nn.relu(_dot(fm, wf1_r[...]) + bf1_r[...]), wf2_r[...])
        + bf2_r[...])
    xn = x_r[...] + jax.nn.relu(
        _dot(a2a, wca_r[...]) + _dot(f2a, wcf_r[...]) + bc_r[...])
    xn_r[...] = xn
    y_r[...] = jax.nn.relu(_dot(xn, wy_r[...]) + by_r[...])

  g = N // blk
  dspec = [
      pl.BlockSpec((blk, H), lambda i: (i, 0)),
      pl.BlockSpec((2, blk, H), lambda i: (0, i, 0)),
      pl.BlockSpec((blk, 16), lambda i: (i, 0)),
      pl.BlockSpec((blk, 1), lambda i: (i, 0)),
      pl.BlockSpec((2, blk, H), lambda i: (0, i, 0)),
      pl.BlockSpec((blk, 1), lambda i: (i, 0)),
  ]
  wspec = [_full(w.shape) for w in wts]
  return pl.pallas_call(
      body,
      grid=(g,),
      in_specs=dspec + wspec,
      out_specs=[pl.BlockSpec((blk, H), lambda i: (i, 0))] * 2,
      out_shape=[jax.ShapeDtypeStruct((N, H), F32)] * 2,
  )(x, accx, histn, rdega, accf, rdegf, *wts)


def _tc_frag_layer(xf, acc_a2f, acc_f2f, rca, rcf, wts):
  def body(xf_r, aa_r, af_r, rca_r, rcf_r, wa1_r, ba1_r, wa2_r, ba2_r,
           wff1_r, bff1_r, wff2_r, bff2_r, wcf_r, wca_r, bcf_r, o_r):
    a2f_m = (aa_r[0] + aa_r[1]) * rca_r[...]
    f2f_m = (af_r[0] + af_r[1]) * rcf_r[...]
    a2f = jax.nn.relu(
        _dot(jax.nn.relu(_dot(a2f_m, wa1_r[...]) + ba1_r[...]), wa2_r[...])
        + ba2_r[...])
    f2f = jax.nn.relu(
        _dot(jax.nn.relu(_dot(f2f_m, wff1_r[...]) + bff1_r[...]), wff2_r[...])
        + bff2_r[...])
    o_r[...] = xf_r[...] + jax.nn.relu(
        _dot(f2f, wcf_r[...]) + _dot(a2f, wca_r[...]) + bcf_r[...])

  specs = ([_full((NF, H)), _full((2, NF, H)), _full((2, NF, H)),
            _full((NF, 1)), _full((NF, 1))]
           + [_full(w.shape) for w in wts])
  return pl.pallas_call(
      body,
      in_specs=specs,
      out_specs=_full((NF, H)),
      out_shape=jax.ShapeDtypeStruct((NF, H), F32),
  )(xf, acc_a2f, acc_f2f, rca, rcf, *wts)


def _tc_seg(x, b2d, n, blk):
  def body(x_r, b_r, s_r, c_r):
    i = pl.program_id(0)

    @pl.when(i == 0)
    def _():
      s_r[...] = jnp.zeros_like(s_r)
      c_r[...] = jnp.zeros_like(c_r)

    io = lax.broadcasted_iota(jnp.int32, (1, BSZ), 1).astype(F32)
    oh = (b_r[...] == io).astype(F32)
    s_r[...] += lax.dot_general(oh, x_r[...], (((0,), (0,)), ((), ())),
                                preferred_element_type=F32)
    c_r[...] += lax.dot_general(oh, jnp.ones((blk, 1), F32),
                                (((0,), (0,)), ((), ())),
                                preferred_element_type=F32)

  return pl.pallas_call(
      body,
      grid=(n // blk,),
      in_specs=[pl.BlockSpec((blk, H), lambda i: (i, 0)),
                pl.BlockSpec((blk, 1), lambda i: (i, 0))],
      out_specs=[_full((BSZ, H)), _full((BSZ, 1))],
      out_shape=[jax.ShapeDtypeStruct((BSZ, H), F32),
                 jax.ShapeDtypeStruct((BSZ, 1), F32)],
  )(x, b2d)


def _tc_final(sx, cx, sf, cf, wts):
  def body(sx_r, cx_r, sf_r, cf_r, wa1_r, ba1_r, wa2_r, ba2_r,
           wf1_r, bf1_r, wf2_r, bf2_r, wo_r, bo_r, o_r):
    mx = sx_r[...] * (1.0 / jnp.maximum(cx_r[...], 1.0))
    mf = sf_r[...] * (1.0 / jnp.maximum(cf_r[...], 1.0))
    xg = jax.nn.relu(
        _dot(jax.nn.relu(_dot(mx, wa1_r[...]) + ba1_r[...]), wa2_r[...])
        + ba2_r[...])
    xf = jax.nn.relu(
        _dot(jax.nn.relu(_dot(mf, wf1_r[...]) + bf1_r[...]), wf2_r[...])
        + bf2_r[...])
    o_r[...] = _dot(xg + xf, wo_r[...]) + bo_r[...]

  specs = ([_full((BSZ, H)), _full((BSZ, 1)), _full((BSZ, H)), _full((BSZ, 1))]
           + [_full(w.shape) for w in wts])
  return pl.pallas_call(
      body,
      in_specs=specs,
      out_specs=_full((BSZ, 1)),
      out_shape=jax.ShapeDtypeStruct((BSZ, 1), F32),
  )(sx, cx, sf, cf, *wts)


def kernel(params, x_atoms, edge_index, edge_attr, fragment_types,
           frag_row, frag_col, higher_edge_index, batch, fragments_batch):
  row_e, col_e = edge_index[0], edge_index[1]
  he0, he1 = higher_edge_index[0], higher_edge_index[1]
  iota_n = jnp.arange(N, dtype=jnp.int32)
  iota_nf = jnp.arange(NF, dtype=jnp.int32)

  erow = jnp.tile(jnp.eye(1, H, dtype=F32), (CHUNK, 1))
  HREP = 256
  hist_tab = jnp.tile(jnp.eye(16, H, dtype=F32), (HREP, 1))
  hist_g = ((jnp.arange(E, dtype=jnp.int32) % HREP) * 16
            + edge_attr.astype(jnp.int32))
  embx_p, embf_p, cnta_p, degf_p, cntf_p, hist_p = _sc_fused([
      [(params["atom_emb"], x_atoms.astype(jnp.int32), iota_n, N,
        "replicate"),
       (params["frag_emb"], fragment_types.astype(jnp.int32), iota_nf, NF,
        "replicate"),
       (erow, None, frag_col, NF, "const")],
      [(erow, None, frag_row, N, "const"),
       (erow, None, he1, NF, "const")],
      [(hist_tab, hist_g, col_e, N, "gather")],
  ])
  x = _tc_add2(embx_p[:, :N, :], N)
  (accx_p,) = _sc_fused([[(x, row_e, col_e, N, "gather")]])
  x_frag = _tc_add2(embf_p[:, :NF, :], NF)
  histn, rdeg_a, rdeg_f = _tc_prep_atom(hist_p[:, :N, :16],
                                        degf_p[:, :N, 0:1])
  rc_a2f, rc_f2f = _tc_prep_frag(cnta_p[:, :NF, 0:1], cntf_p[:, :NF, 0:1])

  acc_a2f_p = None
  for li in range(NUM_LAYERS):
    p = params["layers"][li]
    a1, a2 = p["a2a_after"]
    f1, f2 = p["f2a_after"]
    wts_atom = (
        p["bond_emb"],
        a1["w"][:H, :], a1["w"][H:, :], a1["b"].reshape(1, H),
        a2["w"], a2["b"].reshape(1, H),
        f1["w"], f1["b"].reshape(1, H), f2["w"], f2["b"].reshape(1, H),
        p["combine_atom"][0]["w"][:H, :], p["combine_atom"][0]["w"][H:, :],
        p["combine_atom"][0]["b"].reshape(1, H),
        p["a2f_before"][0]["w"], p["a2f_before"][0]["b"].reshape(1, H),
    )
    accf_p, accff_p = _sc_fused([[(x_frag, frag_col, frag_row, N, "gather"),
                                  (x_frag, he0, he1, NF, "gather")]])
    x, y = _tc_atom_layer(x, accx_p[:, :N, :], histn, rdeg_a,
                          accf_p[:, :N, :], rdeg_f, wts_atom)
    if li < NUM_LAYERS - 1:
      acc_a2f_p, accx_p = _sc_fused([[(y, frag_row, frag_col, NF, "gather"),
                                      (x, row_e, col_e, N, "gather")]])
    else:
      (acc_a2f_p,) = _sc_fused([[(y, frag_row, frag_col, NF, "gather")]])

    q1, q2 = p["a2f_after"]
    r1, r2 = p["f2f_after"]
    wts_frag = (
        q1["w"], q1["b"].reshape(1, H), q2["w"], q2["b"].reshape(1, H),
        r1["w"], r1["b"].reshape(1, H), r2["w"], r2["b"].reshape(1, H),
        p["combine_frag"][0]["w"][:H, :], p["combine_frag"][0]["w"][H:, :],
        p["combine_frag"][0]["b"].reshape(1, H),
    )
    x_frag = _tc_frag_layer(x_frag, acc_a2f_p[:, :NF, :], accff_p[:, :NF, :],
                            rc_a2f, rc_f2f, wts_frag)

  sx, cx = _tc_seg(x, batch.astype(F32).reshape(N, 1), N, 2000)
  sf, cf = _tc_seg(x_frag, fragments_batch.astype(F32).reshape(NF, 1), NF, NF)
  ao1, ao2 = params["atom_out"]
  fo1, fo2 = params["frag_out"]
  wts_fin = (ao1["w"], ao1["b"].reshape(1, H), ao2["w"], ao2["b"].reshape(1, H),
             fo1["w"], fo1["b"].reshape(1, H), fo2["w"], fo2["b"].reshape(1, H),
             params["out"][0]["w"], params["out"][0]["b"].reshape(1, 1))
  return _tc_final(sx, cx, sf, cf, wts_fin)
